# baseline (speedup 1.0000x reference)
import functools

import jax
import jax.numpy as jnp
from jax import lax
from jax.experimental import pallas as pl
from jax.experimental.pallas import tpu as pltpu
from jax.experimental.pallas import tpu_sc as plsc


def _proj_body(x_ref, w_ref, out_ref):
    out_ref[...] = jnp.dot(x_ref[...], w_ref[...],
                           preferred_element_type=jnp.float32)


def _make_scatter16(n, npad, h, e, nc, ns, ch):
    nw = nc * ns
    epw = e // nw
    nchunk = epw // ch
    rps = npad // ns
    nfull = n // rps
    nrem = n % rps
    mesh = plsc.VectorSubcoreMesh(core_axis_name="c", subcore_axis_name="s")

    @functools.partial(
        pl.kernel,
        out_type=jax.ShapeDtypeStruct((nc, npad, h), jnp.float32),
        mesh=mesh,
        scratch_types=[
            pltpu.VMEM((2, ch), jnp.int32),
            pltpu.VMEM((2, ch), jnp.int32),
            pltpu.VMEM((2, ch, h), jnp.float32),
            pltpu.VMEM((rps, h), jnp.float32),
            pltpu.VMEM_SHARED((npad, h), jnp.float32),
            pltpu.SemaphoreType.DMA,
            pltpu.SemaphoreType.DMA,
        ],
        compiler_params=pltpu.CompilerParams(use_tc_tiling_on_sc=False),
    )
    def scatter16(p_hbm, edge_hbm, out_hbm, src_v, dst_v, rows_v,
                  stage_v, accum, sem0, sem1):
        cid = lax.axis_index("c")
        sid = lax.axis_index("s")
        wid = sid * nc + cid
        sems = (sem0, sem1)

        def _zrow(i, _):
            stage_v[i] = jnp.zeros((h,), jnp.float32)
            return 0
        lax.fori_loop(0, rps, _zrow, 0)

        @pl.when(jnp.logical_and(cid == 0, sid < nfull))
        def _():
            pltpu.sync_copy(p_hbm.at[pl.ds(sid * rps, rps)], stage_v)

        if nrem > 0:
            @pl.when(jnp.logical_and(cid == 0, sid == nfull))
            def _():
                pltpu.sync_copy(p_hbm.at[pl.ds(nfull * rps, nrem)],
                                stage_v.at[pl.ds(0, nrem)])

        pltpu.sync_copy(stage_v, accum.at[pl.ds(sid * rps, rps)])
        plsc.subcore_barrier()

        base0 = wid * epw
        pltpu.sync_copy(edge_hbm.at[0, pl.ds(base0, ch)], src_v.at[0])
        pltpu.sync_copy(edge_hbm.at[1, pl.ds(base0, ch)], dst_v.at[0])
        g = pltpu.async_copy(p_hbm.at[src_v.at[0]], rows_v.at[0], sems[0])
        for k in range(nchunk):
            b = k % 2
            nb = (k + 1) % 2
            if k + 1 < nchunk:
                basek = wid * epw + (k + 1) * ch
                pltpu.sync_copy(edge_hbm.at[0, pl.ds(basek, ch)],
                                src_v.at[nb])
                pltpu.sync_copy(edge_hbm.at[1, pl.ds(basek, ch)],
                                dst_v.at[nb])
                g.wait()
                g = pltpu.async_copy(p_hbm.at[src_v.at[nb]], rows_v.at[nb],
                                     sems[nb])
            else:
                g.wait()
            pltpu.sync_copy(rows_v.at[b], accum.at[dst_v.at[b]], add=True)
        plsc.subcore_barrier()

        pltpu.sync_copy(accum.at[pl.ds(sid * rps, rps)], stage_v)
        pltpu.sync_copy(stage_v, out_hbm.at[cid, pl.ds(sid * rps, rps)])

    return scatter16


def _rsqrt_newton(x):
    i = plsc.bitcast(x, jnp.int32)
    y = plsc.bitcast(jnp.int32(0x5F3759DF) - (i >> 1), jnp.float32)
    for _ in range(3):
        y = y * (1.5 - 0.5 * x * y * y)
    return y


def _make_middle(n, npad, h, e, nc, ns, ch):
    nw = nc * ns
    epw = e // nw
    nchunk = epw // ch
    rs = npad // ns
    fl = rs * h
    half = npad // nc
    hps = half // ns
    epc = e // ns
    cap = epc
    mesh = plsc.VectorSubcoreMesh(core_axis_name="c", subcore_axis_name="s")

    @functools.partial(
        pl.kernel,
        out_type=jax.ShapeDtypeStruct((n,), jnp.float32),
        mesh=mesh,
        scratch_types=[
            pltpu.VMEM((fl,), jnp.float32),
            pltpu.VMEM((fl,), jnp.float32),
            pltpu.VMEM((h,), jnp.float32),
            pltpu.VMEM((h,), jnp.float32),
            pltpu.VMEM((ns * 2 * h,), jnp.float32),
            pltpu.VMEM((4 * h,), jnp.float32),
            pltpu.VMEM((rs,), jnp.float32),
            pltpu.VMEM((hps,), jnp.float32),
            pltpu.VMEM((cap,), jnp.int32),
            pltpu.VMEM((cap,), jnp.int32),
            pltpu.VMEM((cap,), jnp.float32),
            pltpu.VMEM_SHARED((ns * 2 * h,), jnp.float32),
            pltpu.VMEM_SHARED((npad,), jnp.float32),
            pltpu.VMEM_SHARED((half + 64,), jnp.float32),
            pltpu.SemaphoreType.DMA,
        ],
        compiler_params=pltpu.CompilerParams(use_tc_tiling_on_sc=False,
                                             needs_layout_passes=False),
    )
    def middle(parts_hbm, gbw_hbm, edge_hbm, out_hbm,
               pa_v, agg_v, ssum_v, ssq_v, allst_v, gbw_v, q_v, zb_v,
               cs_v, cd_v, vals_v, stats_sp, qsp, accq, sem0):
        cid = lax.axis_index("c")
        sid = lax.axis_index("s")
        wid = sid * nc + cid

        pltpu.sync_copy(parts_hbm.at[0, pl.ds(sid * fl, fl)], pa_v)
        pltpu.sync_copy(parts_hbm.at[1, pl.ds(sid * fl, fl)], agg_v)
        pltpu.sync_copy(gbw_hbm, gbw_v)

        def _p1(r, carry):
            s, sq = carry
            v = pa_v[pl.ds(r * h, h)] + agg_v[pl.ds(r * h, h)]
            agg_v[pl.ds(r * h, h)] = v
            return (s + v, sq + v * v)
        zs = jnp.zeros((h,), jnp.float32)
        s_part, sq_part = lax.fori_loop(0, rs, _p1, (zs, zs))
        ssum_v[...] = s_part
        ssq_v[...] = sq_part
        pltpu.sync_copy(ssum_v, stats_sp.at[pl.ds(sid * 2 * h, h)])
        pltpu.sync_copy(ssq_v, stats_sp.at[pl.ds(sid * 2 * h + h, h)])
        plsc.subcore_barrier()

        pltpu.sync_copy(stats_sp, allst_v)

        def _p2(i, carry):
            s, sq = carry
            return (s + allst_v[pl.ds(i * 2 * h, h)],
                    sq + allst_v[pl.ds(i * 2 * h + h, h)])
        s_all, sq_all = lax.fori_loop(0, ns, _p2, (zs, zs))
        mu = s_all * (1.0 / n)
        var = sq_all * (1.0 / n) - mu * mu
        inv = _rsqrt_newton(var + 1e-5)
        scale = inv * gbw_v[pl.ds(0, h)]
        shift = gbw_v[pl.ds(h, h)] - mu * scale
        w2 = gbw_v[pl.ds(2 * h, h)]

        lanes = lax.iota(jnp.int32, 16)

        def _p3(g, _):
            def _row(j, qvec):
                v = agg_v[pl.ds((g * 16 + j) * h, h)] * scale + shift
                v = jnp.where(v > 0, v, jnp.exp(v) - 1.0)
                return jnp.where(lanes == j, jnp.sum(v * w2), qvec)
            qvec = lax.fori_loop(0, 16, _row, jnp.zeros((16,), jnp.float32))
            q_v[pl.ds(g * 16, 16)] = qvec
            return 0
        lax.fori_loop(0, rs // 16, _p3, 0)

        pltpu.sync_copy(q_v, qsp.at[pl.ds(sid * rs, rs)])
        plsc.subcore_barrier()

        pltpu.sync_copy(qsp.at[pl.ds(cid * half + sid * hps, hps)], zb_v)
        pltpu.sync_copy(zb_v, accq.at[pl.ds(sid * hps, hps)])

        @pl.when(sid == 0)
        def _():
            for j in range(4):
                vals_v[pl.ds(j * 16, 16)] = jnp.zeros((16,), jnp.float32)
            pltpu.sync_copy(vals_v.at[pl.ds(0, 64)],
                            accq.at[pl.ds(half, 64)])
        plsc.subcore_barrier()

        lo = cid * half
        pltpu.sync_copy(edge_hbm.at[0, pl.ds(sid * epc, epc)], cs_v)
        pltpu.sync_copy(edge_hbm.at[1, pl.ds(sid * epc, epc)], cd_v)

        def _redir(i, _):
            dl = cd_v[pl.ds(i * 16, 16)] - lo
            m = jnp.logical_and(dl >= 0, dl < half)
            sen = half + ((lanes + i * 16) & 63)
            cd_v[pl.ds(i * 16, 16)] = jnp.where(m, dl, sen)
            return 0
        lax.fori_loop(0, epc // 16, _redir, 0)

        pltpu.async_copy(qsp.at[cs_v], vals_v, sem0).wait()
        pltpu.sync_copy(vals_v, accq.at[cd_v], add=True)
        plsc.subcore_barrier()

        b2v = gbw_v[pl.ds(3 * h, h)]
        pltpu.sync_copy(accq.at[pl.ds(sid * hps, hps)], zb_v)

        def _tanh(i, _):
            z = zb_v[pl.ds(i * 16, 16)] + b2v
            zb_v[pl.ds(i * 16, 16)] = 1.0 - 2.0 / (jnp.exp(2.0 * z) + 1.0)
            return 0
        lax.fori_loop(0, hps // 16, _tanh, 0)

        g0 = cid * half + sid * hps

        @pl.when(g0 + hps <= n)
        def _():
            pltpu.sync_copy(zb_v, out_hbm.at[pl.ds(g0, hps)])

        rem = n % hps
        if rem > 0:
            @pl.when(jnp.logical_and(g0 < n, g0 + hps > n))
            def _():
                pltpu.sync_copy(zb_v.at[pl.ds(0, rem)],
                                out_hbm.at[pl.ds(n - rem, rem)])

    return middle


def kernel(x, edge_index, W1, b1, gamma, beta, W2, b2):
    n, d = x.shape
    hdim = W1.shape[1]
    e = edge_index.shape[1]
    odim = W2.shape[1]

    info = plsc.get_sparse_core_info()
    nc, ns = info.num_cores, info.num_subcores

    p = pl.pallas_call(
        _proj_body,
        out_shape=jax.ShapeDtypeStruct((n, hdim), jnp.float32),
    )(x, W1)

    npad = ((n + 16 * ns - 1) // (16 * ns)) * (16 * ns)

    parts = _make_scatter16(n, npad, hdim, e, nc, ns, ch=2000)(p, edge_index)

    gbw = jnp.concatenate([gamma, beta, W2.reshape(hdim),
                           jnp.broadcast_to(b2, (hdim,))])
    out_flat = _make_middle(n, npad, hdim, e, nc, ns, ch=2000)(
        parts.reshape(nc, npad * hdim), gbw, edge_index)

    return out_flat.reshape(n, odim)


# speedup vs baseline: 1.1053x; 1.1053x over previous
import functools

import jax
import jax.numpy as jnp
from jax import lax
from jax.experimental import pallas as pl
from jax.experimental.pallas import tpu as pltpu
from jax.experimental.pallas import tpu_sc as plsc


def _proj_body(x_ref, w_ref, out_ref):
    out_ref[...] = jnp.dot(x_ref[...], w_ref[...],
                           preferred_element_type=jnp.float32)


def _tail_body(qparts_ref, b2_ref, out_ref):
    n = out_ref.shape[0]
    out_ref[...] = jnp.tanh(qparts_ref[0, :n] + qparts_ref[1, :n]
                            + b2_ref[0])


def _make_scatter16(n, npad, h, e, nc, ns, ch):
    nw = nc * ns
    epw = e // nw
    nchunk = epw // ch
    rps = npad // ns
    nfull = n // rps
    nrem = n % rps
    mesh = plsc.VectorSubcoreMesh(core_axis_name="c", subcore_axis_name="s")

    @functools.partial(
        pl.kernel,
        out_type=jax.ShapeDtypeStruct((nc, npad, h), jnp.float32),
        mesh=mesh,
        scratch_types=[
            pltpu.VMEM((2, ch), jnp.int32),
            pltpu.VMEM((2, ch), jnp.int32),
            pltpu.VMEM((2, ch, h), jnp.float32),
            pltpu.VMEM((rps, h), jnp.float32),
            pltpu.VMEM_SHARED((npad, h), jnp.float32),
            pltpu.SemaphoreType.DMA,
            pltpu.SemaphoreType.DMA,
        ],
        compiler_params=pltpu.CompilerParams(use_tc_tiling_on_sc=False),
    )
    def scatter16(p_hbm, edge_hbm, out_hbm, src_v, dst_v, rows_v,
                  stage_v, accum, sem0, sem1):
        cid = lax.axis_index("c")
        sid = lax.axis_index("s")
        wid = sid * nc + cid
        sems = (sem0, sem1)

        def _zrow(i, _):
            stage_v[i] = jnp.zeros((h,), jnp.float32)
            return 0
        lax.fori_loop(0, rps, _zrow, 0)

        @pl.when(jnp.logical_and(cid == 0, sid < nfull))
        def _():
            pltpu.sync_copy(p_hbm.at[pl.ds(sid * rps, rps)], stage_v)

        if nrem > 0:
            @pl.when(jnp.logical_and(cid == 0, sid == nfull))
            def _():
                pltpu.sync_copy(p_hbm.at[pl.ds(nfull * rps, nrem)],
                                stage_v.at[pl.ds(0, nrem)])

        pltpu.sync_copy(stage_v, accum.at[pl.ds(sid * rps, rps)])
        plsc.subcore_barrier()

        base0 = wid * epw
        pltpu.sync_copy(edge_hbm.at[0, pl.ds(base0, ch)], src_v.at[0])
        pltpu.sync_copy(edge_hbm.at[1, pl.ds(base0, ch)], dst_v.at[0])
        g = pltpu.async_copy(p_hbm.at[src_v.at[0]], rows_v.at[0], sems[0])
        for k in range(nchunk):
            b = k % 2
            nb = (k + 1) % 2
            if k + 1 < nchunk:
                basek = wid * epw + (k + 1) * ch
                pltpu.sync_copy(edge_hbm.at[0, pl.ds(basek, ch)],
                                src_v.at[nb])
                pltpu.sync_copy(edge_hbm.at[1, pl.ds(basek, ch)],
                                dst_v.at[nb])
                g.wait()
                g = pltpu.async_copy(p_hbm.at[src_v.at[nb]], rows_v.at[nb],
                                     sems[nb])
            else:
                g.wait()
            pltpu.sync_copy(rows_v.at[b], accum.at[dst_v.at[b]], add=True)
        plsc.subcore_barrier()

        pltpu.sync_copy(accum.at[pl.ds(sid * rps, rps)], stage_v)
        pltpu.sync_copy(stage_v, out_hbm.at[cid, pl.ds(sid * rps, rps)])

    return scatter16


def _rsqrt_newton(x):
    i = plsc.bitcast(x, jnp.int32)
    y = plsc.bitcast(jnp.int32(0x5F3759DF) - (i >> 1), jnp.float32)
    for _ in range(3):
        y = y * (1.5 - 0.5 * x * y * y)
    return y


def _make_middle(n, npad, h, e, nc, ns, ch):
    nw = nc * ns
    epw = e // nw
    nchunk = epw // ch
    rs = npad // ns
    fl = rs * h
    mesh = plsc.VectorSubcoreMesh(core_axis_name="c", subcore_axis_name="s")

    @functools.partial(
        pl.kernel,
        out_type=jax.ShapeDtypeStruct((nc, npad), jnp.float32),
        mesh=mesh,
        scratch_types=[
            pltpu.VMEM((fl,), jnp.float32),
            pltpu.VMEM((fl,), jnp.float32),
            pltpu.VMEM((h,), jnp.float32),
            pltpu.VMEM((h,), jnp.float32),
            pltpu.VMEM((ns * 2 * h,), jnp.float32),
            pltpu.VMEM((3 * h,), jnp.float32),
            pltpu.VMEM((rs,), jnp.float32),
            pltpu.VMEM((rs,), jnp.float32),
            pltpu.VMEM((2, ch), jnp.int32),
            pltpu.VMEM((2, ch), jnp.int32),
            pltpu.VMEM((2, ch), jnp.float32),
            pltpu.VMEM_SHARED((ns * 2 * h,), jnp.float32),
            pltpu.VMEM_SHARED((npad,), jnp.float32),
            pltpu.VMEM_SHARED((npad,), jnp.float32),
            pltpu.SemaphoreType.DMA,
            pltpu.SemaphoreType.DMA,
        ],
        compiler_params=pltpu.CompilerParams(use_tc_tiling_on_sc=False,
                                             needs_layout_passes=False),
    )
    def middle(parts_hbm, gbw_hbm, edge_hbm, out_hbm,
               pa_v, agg_v, ssum_v, ssq_v, allst_v, gbw_v, q_v, zb_v,
               src_v, dst_v, vals_v, stats_sp, qsp, accq, sem0, sem1):
        cid = lax.axis_index("c")
        sid = lax.axis_index("s")
        wid = sid * nc + cid
        sems = (sem0, sem1)

        pltpu.sync_copy(parts_hbm.at[0, pl.ds(sid * fl, fl)], pa_v)
        pltpu.sync_copy(parts_hbm.at[1, pl.ds(sid * fl, fl)], agg_v)
        pltpu.sync_copy(gbw_hbm, gbw_v)

        def _p1(r, carry):
            s, sq = carry
            v = pa_v[pl.ds(r * h, h)] + agg_v[pl.ds(r * h, h)]
            agg_v[pl.ds(r * h, h)] = v
            return (s + v, sq + v * v)
        zs = jnp.zeros((h,), jnp.float32)
        s_part, sq_part = lax.fori_loop(0, rs, _p1, (zs, zs))
        ssum_v[...] = s_part
        ssq_v[...] = sq_part
        pltpu.sync_copy(ssum_v, stats_sp.at[pl.ds(sid * 2 * h, h)])
        pltpu.sync_copy(ssq_v, stats_sp.at[pl.ds(sid * 2 * h + h, h)])
        plsc.subcore_barrier()

        pltpu.sync_copy(stats_sp, allst_v)

        def _p2(i, carry):
            s, sq = carry
            return (s + allst_v[pl.ds(i * 2 * h, h)],
                    sq + allst_v[pl.ds(i * 2 * h + h, h)])
        s_all, sq_all = lax.fori_loop(0, ns, _p2, (zs, zs))
        mu = s_all * (1.0 / n)
        var = sq_all * (1.0 / n) - mu * mu
        inv = _rsqrt_newton(var + 1e-5)
        scale = inv * gbw_v[pl.ds(0, h)]
        shift = gbw_v[pl.ds(h, h)] - mu * scale
        w2 = gbw_v[pl.ds(2 * h, h)]

        lanes = lax.iota(jnp.int32, 16)

        def _p3(g, _):
            def _row(j, qvec):
                v = agg_v[pl.ds((g * 16 + j) * h, h)] * scale + shift
                v = jnp.where(v > 0, v, jnp.exp(v) - 1.0)
                return jnp.where(lanes == j, jnp.sum(v * w2), qvec)
            qvec = lax.fori_loop(0, 16, _row, jnp.zeros((16,), jnp.float32))
            q_v[pl.ds(g * 16, 16)] = qvec
            return 0
        lax.fori_loop(0, rs // 16, _p3, 0)

        def _zvec(i, _):
            zb_v[pl.ds(i * 16, 16)] = jnp.zeros((16,), jnp.float32)
            return 0
        lax.fori_loop(0, rs // 16, _zvec, 0)

        pltpu.sync_copy(q_v, qsp.at[pl.ds(sid * rs, rs)])

        @pl.when(cid == 0)
        def _():
            pltpu.sync_copy(q_v, accq.at[pl.ds(sid * rs, rs)])

        @pl.when(cid != 0)
        def _():
            pltpu.sync_copy(zb_v, accq.at[pl.ds(sid * rs, rs)])

        plsc.subcore_barrier()

        base0 = wid * epw
        pltpu.sync_copy(edge_hbm.at[0, pl.ds(base0, ch)], src_v.at[0])
        pltpu.sync_copy(edge_hbm.at[1, pl.ds(base0, ch)], dst_v.at[0])
        g = pltpu.async_copy(qsp.at[src_v.at[0]], vals_v.at[0], sems[0])
        for k in range(nchunk):
            b = k % 2
            nb = (k + 1) % 2
            if k + 1 < nchunk:
                basek = wid * epw + (k + 1) * ch
                pltpu.sync_copy(edge_hbm.at[0, pl.ds(basek, ch)],
                                src_v.at[nb])
                pltpu.sync_copy(edge_hbm.at[1, pl.ds(basek, ch)],
                                dst_v.at[nb])
                g.wait()
                g = pltpu.async_copy(qsp.at[src_v.at[nb]], vals_v.at[nb],
                                     sems[nb])
            else:
                g.wait()
            pltpu.sync_copy(vals_v.at[b], accq.at[dst_v.at[b]], add=True)
        plsc.subcore_barrier()

        pltpu.sync_copy(accq.at[pl.ds(sid * rs, rs)], zb_v)
        pltpu.sync_copy(zb_v, out_hbm.at[cid, pl.ds(sid * rs, rs)])

    return middle


def kernel(x, edge_index, W1, b1, gamma, beta, W2, b2):
    n, d = x.shape
    hdim = W1.shape[1]
    e = edge_index.shape[1]
    odim = W2.shape[1]

    info = plsc.get_sparse_core_info()
    nc, ns = info.num_cores, info.num_subcores

    p = pl.pallas_call(
        _proj_body,
        out_shape=jax.ShapeDtypeStruct((n, hdim), jnp.float32),
    )(x, W1)

    npad = ((n + 16 * ns - 1) // (16 * ns)) * (16 * ns)

    parts = _make_scatter16(n, npad, hdim, e, nc, ns, ch=2000)(p, edge_index)

    gbw = jnp.concatenate([gamma, beta, W2.reshape(hdim)])
    qparts = _make_middle(n, npad, hdim, e, nc, ns, ch=2000)(
        parts.reshape(nc, npad * hdim), gbw, edge_index)

    out_flat = pl.pallas_call(
        _tail_body,
        out_shape=jax.ShapeDtypeStruct((n,), jnp.float32),
    )(qparts, b2)

    return out_flat.reshape(n, odim)


# speedup vs baseline: 1.1243x; 1.0172x over previous
import functools

import jax
import jax.numpy as jnp
from jax import lax
from jax.experimental import pallas as pl
from jax.experimental.pallas import tpu as pltpu
from jax.experimental.pallas import tpu_sc as plsc


def _proj_body(x_ref, w_ref, out_ref):
    out_ref[...] = jnp.dot(x_ref[...], w_ref[...],
                           preferred_element_type=jnp.float32)


def _tail_body(qparts_ref, b2_ref, out_ref):
    n = out_ref.shape[0]
    out_ref[...] = jnp.tanh(qparts_ref[0, :n] + qparts_ref[1, :n]
                            + b2_ref[0])


def _make_scatter16(n, npad, h, e, nc, ns, ch):
    nw = nc * ns
    epw = e // nw
    nchunk = epw // ch
    rps = npad // ns
    nfull = n // rps
    nrem = n % rps
    mesh = plsc.VectorSubcoreMesh(core_axis_name="c", subcore_axis_name="s")

    @functools.partial(
        pl.kernel,
        out_type=jax.ShapeDtypeStruct((nc, npad, h), jnp.float32),
        mesh=mesh,
        scratch_types=[
            pltpu.VMEM((2, ch), jnp.int32),
            pltpu.VMEM((2, ch), jnp.int32),
            pltpu.VMEM((2, ch, h), jnp.float32),
            pltpu.VMEM((rps, h), jnp.float32),
            pltpu.VMEM_SHARED((npad, h), jnp.float32),
            pltpu.SemaphoreType.DMA,
            pltpu.SemaphoreType.DMA,
        ],
        compiler_params=pltpu.CompilerParams(use_tc_tiling_on_sc=False),
    )
    def scatter16(p_hbm, edge_hbm, out_hbm, src_v, dst_v, rows_v,
                  stage_v, accum, sem0, sem1):
        cid = lax.axis_index("c")
        sid = lax.axis_index("s")
        wid = sid * nc + cid
        sems = (sem0, sem1)

        base0 = wid * epw
        pltpu.sync_copy(edge_hbm.at[0, pl.ds(base0, ch)], src_v.at[0])
        pltpu.sync_copy(edge_hbm.at[1, pl.ds(base0, ch)], dst_v.at[0])
        g = pltpu.async_copy(p_hbm.at[src_v.at[0]], rows_v.at[0], sems[0])

        def _zrow(i, _):
            stage_v[i] = jnp.zeros((h,), jnp.float32)
            return 0
        lax.fori_loop(0, rps, _zrow, 0)

        @pl.when(jnp.logical_and(cid == 0, sid < nfull))
        def _():
            pltpu.sync_copy(p_hbm.at[pl.ds(sid * rps, rps)], stage_v)

        if nrem > 0:
            @pl.when(jnp.logical_and(cid == 0, sid == nfull))
            def _():
                pltpu.sync_copy(p_hbm.at[pl.ds(nfull * rps, nrem)],
                                stage_v.at[pl.ds(0, nrem)])

        pltpu.sync_copy(stage_v, accum.at[pl.ds(sid * rps, rps)])
        plsc.subcore_barrier()

        for k in range(nchunk):
            b = k % 2
            nb = (k + 1) % 2
            if k + 1 < nchunk:
                basek = wid * epw + (k + 1) * ch
                pltpu.sync_copy(edge_hbm.at[0, pl.ds(basek, ch)],
                                src_v.at[nb])
                pltpu.sync_copy(edge_hbm.at[1, pl.ds(basek, ch)],
                                dst_v.at[nb])
                g.wait()
                g = pltpu.async_copy(p_hbm.at[src_v.at[nb]], rows_v.at[nb],
                                     sems[nb])
            else:
                g.wait()
            pltpu.sync_copy(rows_v.at[b], accum.at[dst_v.at[b]], add=True)
        plsc.subcore_barrier()

        pltpu.sync_copy(accum.at[pl.ds(sid * rps, rps)], stage_v)
        pltpu.sync_copy(stage_v, out_hbm.at[cid, pl.ds(sid * rps, rps)])

    return scatter16


def _rsqrt_newton(x):
    i = plsc.bitcast(x, jnp.int32)
    y = plsc.bitcast(jnp.int32(0x5F3759DF) - (i >> 1), jnp.float32)
    for _ in range(3):
        y = y * (1.5 - 0.5 * x * y * y)
    return y


def _make_middle(n, npad, h, e, nc, ns, ch):
    nw = nc * ns
    epw = e // nw
    nchunk = epw // ch
    rs = npad // ns
    fl = rs * h
    mesh = plsc.VectorSubcoreMesh(core_axis_name="c", subcore_axis_name="s")

    @functools.partial(
        pl.kernel,
        out_type=jax.ShapeDtypeStruct((nc, npad), jnp.float32),
        mesh=mesh,
        scratch_types=[
            pltpu.VMEM((fl,), jnp.float32),
            pltpu.VMEM((fl,), jnp.float32),
            pltpu.VMEM((h,), jnp.float32),
            pltpu.VMEM((h,), jnp.float32),
            pltpu.VMEM((ns * 2 * h,), jnp.float32),
            pltpu.VMEM((3 * h,), jnp.float32),
            pltpu.VMEM((rs,), jnp.float32),
            pltpu.VMEM((rs,), jnp.float32),
            pltpu.VMEM((2, ch), jnp.int32),
            pltpu.VMEM((2, ch), jnp.int32),
            pltpu.VMEM((2, ch), jnp.float32),
            pltpu.VMEM_SHARED((ns * 2 * h,), jnp.float32),
            pltpu.VMEM_SHARED((npad,), jnp.float32),
            pltpu.VMEM_SHARED((npad,), jnp.float32),
            pltpu.SemaphoreType.DMA,
            pltpu.SemaphoreType.DMA,
        ],
        compiler_params=pltpu.CompilerParams(use_tc_tiling_on_sc=False,
                                             needs_layout_passes=False),
    )
    def middle(parts_hbm, gbw_hbm, edge_hbm, out_hbm,
               pa_v, agg_v, ssum_v, ssq_v, allst_v, gbw_v, q_v, zb_v,
               src_v, dst_v, vals_v, stats_sp, qsp, accq, sem0, sem1):
        cid = lax.axis_index("c")
        sid = lax.axis_index("s")
        wid = sid * nc + cid
        sems = (sem0, sem1)

        base0 = wid * epw
        pltpu.sync_copy(edge_hbm.at[0, pl.ds(base0, ch)], src_v.at[0])
        pltpu.sync_copy(edge_hbm.at[1, pl.ds(base0, ch)], dst_v.at[0])

        pltpu.sync_copy(parts_hbm.at[0, pl.ds(sid * fl, fl)], pa_v)
        pltpu.sync_copy(parts_hbm.at[1, pl.ds(sid * fl, fl)], agg_v)
        pltpu.sync_copy(gbw_hbm, gbw_v)

        def _p1(r, carry):
            s, sq = carry
            v = pa_v[pl.ds(r * h, h)] + agg_v[pl.ds(r * h, h)]
            agg_v[pl.ds(r * h, h)] = v
            return (s + v, sq + v * v)
        zs = jnp.zeros((h,), jnp.float32)
        s_part, sq_part = lax.fori_loop(0, rs, _p1, (zs, zs))
        ssum_v[...] = s_part
        ssq_v[...] = sq_part
        pltpu.sync_copy(ssum_v, stats_sp.at[pl.ds(sid * 2 * h, h)])
        pltpu.sync_copy(ssq_v, stats_sp.at[pl.ds(sid * 2 * h + h, h)])
        plsc.subcore_barrier()

        pltpu.sync_copy(stats_sp, allst_v)

        def _p2(i, carry):
            s, sq = carry
            return (s + allst_v[pl.ds(i * 2 * h, h)],
                    sq + allst_v[pl.ds(i * 2 * h + h, h)])
        s_all, sq_all = lax.fori_loop(0, ns, _p2, (zs, zs))
        mu = s_all * (1.0 / n)
        var = sq_all * (1.0 / n) - mu * mu
        inv = _rsqrt_newton(var + 1e-5)
        scale = inv * gbw_v[pl.ds(0, h)]
        shift = gbw_v[pl.ds(h, h)] - mu * scale
        w2 = gbw_v[pl.ds(2 * h, h)]

        lanes = lax.iota(jnp.int32, 16)

        def _p3(g, _):
            def _row(j, qvec):
                v = agg_v[pl.ds((g * 16 + j) * h, h)] * scale + shift
                v = jnp.where(v > 0, v, jnp.exp(v) - 1.0)
                return jnp.where(lanes == j, jnp.sum(v * w2), qvec)
            qvec = lax.fori_loop(0, 16, _row, jnp.zeros((16,), jnp.float32))
            q_v[pl.ds(g * 16, 16)] = qvec
            return 0
        lax.fori_loop(0, rs // 16, _p3, 0)

        def _zvec(i, _):
            zb_v[pl.ds(i * 16, 16)] = jnp.zeros((16,), jnp.float32)
            return 0
        lax.fori_loop(0, rs // 16, _zvec, 0)

        pltpu.sync_copy(q_v, qsp.at[pl.ds(sid * rs, rs)])

        @pl.when(cid == 0)
        def _():
            pltpu.sync_copy(q_v, accq.at[pl.ds(sid * rs, rs)])

        @pl.when(cid != 0)
        def _():
            pltpu.sync_copy(zb_v, accq.at[pl.ds(sid * rs, rs)])

        plsc.subcore_barrier()

        g = pltpu.async_copy(qsp.at[src_v.at[0]], vals_v.at[0], sems[0])
        for k in range(nchunk):
            b = k % 2
            nb = (k + 1) % 2
            if k + 1 < nchunk:
                basek = wid * epw + (k + 1) * ch
                pltpu.sync_copy(edge_hbm.at[0, pl.ds(basek, ch)],
                                src_v.at[nb])
                pltpu.sync_copy(edge_hbm.at[1, pl.ds(basek, ch)],
                                dst_v.at[nb])
                g.wait()
                g = pltpu.async_copy(qsp.at[src_v.at[nb]], vals_v.at[nb],
                                     sems[nb])
            else:
                g.wait()
            pltpu.sync_copy(vals_v.at[b], accq.at[dst_v.at[b]], add=True)
        plsc.subcore_barrier()

        pltpu.sync_copy(accq.at[pl.ds(sid * rs, rs)], zb_v)
        pltpu.sync_copy(zb_v, out_hbm.at[cid, pl.ds(sid * rs, rs)])

    return middle


def kernel(x, edge_index, W1, b1, gamma, beta, W2, b2):
    n, d = x.shape
    hdim = W1.shape[1]
    e = edge_index.shape[1]
    odim = W2.shape[1]

    info = plsc.get_sparse_core_info()
    nc, ns = info.num_cores, info.num_subcores

    p = pl.pallas_call(
        _proj_body,
        out_shape=jax.ShapeDtypeStruct((n, hdim), jnp.float32),
    )(x, W1)

    npad = ((n + 16 * ns - 1) // (16 * ns)) * (16 * ns)

    parts = _make_scatter16(n, npad, hdim, e, nc, ns, ch=2000)(p, edge_index)

    gbw = jnp.concatenate([gamma, beta, W2.reshape(hdim)])
    qparts = _make_middle(n, npad, hdim, e, nc, ns, ch=2000)(
        parts.reshape(nc, npad * hdim), gbw, edge_index)

    out_flat = pl.pallas_call(
        _tail_body,
        out_shape=jax.ShapeDtypeStruct((n,), jnp.float32),
    )(qparts, b2)

    return out_flat.reshape(n, odim)
